# Initial kernel scaffold; baseline (speedup 1.0000x reference)
import functools

import jax
import jax.numpy as jnp
from jax import lax
from jax.experimental import pallas as pl
from jax.experimental.pallas import tpu as pltpu
from jax.experimental.pallas import tpu_sc as plsc

_N = 10000
_E = 320000
_DIN = 128
_H = 64
_G = 64

_NC = 2
_NS = 16
_NW = _NC * _NS
_ET = _E // _NW
_CH = 80
_NCH = _ET // _CH
_RPT = _N // _NS

_BN = 2000
_GRID = _N // _BN

_built = {}


def _build_sc_kernels():
    mesh = plsc.VectorSubcoreMesh(core_axis_name="c", subcore_axis_name="s")

    @functools.partial(
        pl.kernel,
        mesh=mesh,
        out_type=jax.ShapeDtypeStruct((_NC, _N, _H), jnp.float32),
        scratch_types=[
            pltpu.VMEM((_NCH, _CH), jnp.int32),
            pltpu.VMEM((_NCH, _CH), jnp.int32),
            pltpu.VMEM((_CH, _H), jnp.float32),
            pltpu.VMEM_SHARED((_N, _H), jnp.float32),
        ],
    )
    def edge_agg(y_hbm, src_hbm, dst_hbm, zero_hbm, out_hbm, src_v, dst_v,
                 rows_v, acc):
        cid = lax.axis_index("c")
        sid = lax.axis_index("s")
        wid = cid * _NS + sid
        pltpu.sync_copy(zero_hbm, acc.at[pl.ds(sid * _RPT, _RPT)])
        pltpu.sync_copy(src_hbm.at[wid], src_v)
        pltpu.sync_copy(dst_hbm.at[wid], dst_v)
        plsc.subcore_barrier()

        @pl.loop(0, _NCH)
        def _(j):
            pltpu.sync_copy(y_hbm.at[src_v.at[j]], rows_v)
            pltpu.sync_copy(rows_v, acc.at[dst_v.at[j]], add=True)

        plsc.subcore_barrier()
        pltpu.sync_copy(acc.at[pl.ds(sid * _RPT, _RPT)],
                        out_hbm.at[cid, pl.ds(sid * _RPT, _RPT)])

    @functools.partial(
        pl.kernel,
        mesh=mesh,
        out_type=jax.ShapeDtypeStruct((_NC, _N, 16), jnp.float32),
        scratch_types=[
            pltpu.VMEM((_NCH, _CH), jnp.int32),
            pltpu.VMEM((_CH, 16), jnp.float32),
            pltpu.VMEM_SHARED((_N, 16), jnp.float32),
        ],
    )
    def deg_hist(dst_hbm, zero_hbm, ones_hbm, out_hbm, dst_v, ones_v, acc):
        cid = lax.axis_index("c")
        sid = lax.axis_index("s")
        wid = cid * _NS + sid
        pltpu.sync_copy(zero_hbm, acc.at[pl.ds(sid * _RPT, _RPT)])
        pltpu.sync_copy(ones_hbm, ones_v)
        pltpu.sync_copy(dst_hbm.at[wid], dst_v)
        plsc.subcore_barrier()

        @pl.loop(0, _NCH)
        def _(j):
            pltpu.sync_copy(ones_v, acc.at[dst_v.at[j]], add=True)

        plsc.subcore_barrier()
        pltpu.sync_copy(acc.at[pl.ds(sid * _RPT, _RPT)],
                        out_hbm.at[cid, pl.ds(sid * _RPT, _RPT)])

    return edge_agg, deg_hist


def _mm1_body(x_ref, w_ref, o_ref):
    o_ref[...] = jnp.dot(x_ref[...], w_ref[...],
                         preferred_element_type=jnp.float32)


def _scale_body(degp_ref, xt_ref, y_ref, dis_ref):
    deg = degp_ref[0, :, :1] + degp_ref[1, :, :1] + 1.0
    dis = lax.rsqrt(deg)
    dis_ref[...] = dis
    y_ref[...] = xt_ref[...] * dis


def _mid_body(agg_ref, y_ref, dis_ref, b_ref, w_ref, x1_ref, y2_ref):
    dis = dis_ref[...]
    s = agg_ref[0] + agg_ref[1] + y_ref[...]
    x1 = jnp.tanh(s * dis + b_ref[...])
    x1_ref[...] = x1
    y2_ref[...] = jnp.dot(x1, w_ref[...],
                          preferred_element_type=jnp.float32) * dis


def _final_body(agg_ref, y_ref, dis_ref, b_ref, batch_ref, x2_ref, mean_ref,
                cnt_ref):
    i = pl.program_id(0)

    @pl.when(i == 0)
    def _():
        mean_ref[...] = jnp.zeros_like(mean_ref)
        cnt_ref[...] = jnp.zeros_like(cnt_ref)

    s = agg_ref[0] + agg_ref[1] + y_ref[...]
    x2 = jnp.tanh(s * dis_ref[...] + b_ref[...])
    x2_ref[...] = x2
    onehot = (batch_ref[...] == lax.broadcasted_iota(
        jnp.int32, (_BN, _G), 1)).astype(jnp.float32)
    mean_ref[...] += lax.dot_general(onehot, x2, (((0,), (0,)), ((), ())),
                                     preferred_element_type=jnp.float32)
    cnt_ref[...] += lax.dot_general(onehot, jnp.ones((_BN, 1), jnp.float32),
                                    (((0,), (0,)), ((), ())),
                                    preferred_element_type=jnp.float32)

    @pl.when(i == _GRID - 1)
    def _():
        mean_ref[...] = mean_ref[...] / jnp.maximum(cnt_ref[...], 1.0)


def _tc_matmul1(attrs, W1):
    return pl.pallas_call(
        _mm1_body,
        grid=(_GRID,),
        in_specs=[
            pl.BlockSpec((_BN, _DIN), lambda i: (i, 0)),
            pl.BlockSpec((_DIN, _H), lambda i: (0, 0)),
        ],
        out_specs=pl.BlockSpec((_BN, _H), lambda i: (i, 0)),
        out_shape=jax.ShapeDtypeStruct((_N, _H), jnp.float32),
    )(attrs, W1)


def _tc_scale(deg_p, xt1):
    return pl.pallas_call(
        _scale_body,
        grid=(_GRID,),
        in_specs=[
            pl.BlockSpec((_NC, _BN, 16), lambda i: (0, i, 0)),
            pl.BlockSpec((_BN, _H), lambda i: (i, 0)),
        ],
        out_specs=[
            pl.BlockSpec((_BN, _H), lambda i: (i, 0)),
            pl.BlockSpec((_BN, 1), lambda i: (i, 0)),
        ],
        out_shape=[
            jax.ShapeDtypeStruct((_N, _H), jnp.float32),
            jax.ShapeDtypeStruct((_N, 1), jnp.float32),
        ],
    )(deg_p, xt1)


def _tc_mid(agg, y1, dis, b1r, W2):
    return pl.pallas_call(
        _mid_body,
        grid=(_GRID,),
        in_specs=[
            pl.BlockSpec((_NC, _BN, _H), lambda i: (0, i, 0)),
            pl.BlockSpec((_BN, _H), lambda i: (i, 0)),
            pl.BlockSpec((_BN, 1), lambda i: (i, 0)),
            pl.BlockSpec((1, _H), lambda i: (0, 0)),
            pl.BlockSpec((_H, _H), lambda i: (0, 0)),
        ],
        out_specs=[
            pl.BlockSpec((_BN, _H), lambda i: (i, 0)),
            pl.BlockSpec((_BN, _H), lambda i: (i, 0)),
        ],
        out_shape=[
            jax.ShapeDtypeStruct((_N, _H), jnp.float32),
            jax.ShapeDtypeStruct((_N, _H), jnp.float32),
        ],
    )(agg, y1, dis, b1r, W2)


def _tc_final(agg, y2, dis, b2r, batch2):
    return pl.pallas_call(
        _final_body,
        grid=(_GRID,),
        in_specs=[
            pl.BlockSpec((_NC, _BN, _H), lambda i: (0, i, 0)),
            pl.BlockSpec((_BN, _H), lambda i: (i, 0)),
            pl.BlockSpec((_BN, 1), lambda i: (i, 0)),
            pl.BlockSpec((1, _H), lambda i: (0, 0)),
            pl.BlockSpec((_BN, 1), lambda i: (i, 0)),
        ],
        out_specs=[
            pl.BlockSpec((_BN, _H), lambda i: (i, 0)),
            pl.BlockSpec((_G, _H), lambda i: (0, 0)),
        ],
        out_shape=[
            jax.ShapeDtypeStruct((_N, _H), jnp.float32),
            jax.ShapeDtypeStruct((_G, _H), jnp.float32),
        ],
        scratch_shapes=[pltpu.VMEM((_G, 1), jnp.float32)],
    )(agg, y2, dis, b2r, batch2)


def kernel(attrs, edge_index, batch, W1, b1, W2, b2):
    if "sc" not in _built:
        _built["sc"] = _build_sc_kernels()
    edge_agg, deg_hist = _built["sc"]

    f32 = jnp.float32
    src3 = edge_index[0].reshape(_NW, _NCH, _CH)
    dst3 = edge_index[1].reshape(_NW, _NCH, _CH)
    zero64 = jnp.zeros((_RPT, _H), f32)
    zero16 = jnp.zeros((_RPT, 16), f32)
    ones16 = jnp.ones((_CH, 16), f32)
    b1r = b1.reshape(1, _H)
    b2r = b2.reshape(1, _H)
    batch2 = batch.reshape(_N, 1)

    deg_p = deg_hist(dst3, zero16, ones16)
    xt1 = _tc_matmul1(attrs.astype(f32), W1)
    y1, dis = _tc_scale(deg_p, xt1)
    agg1 = edge_agg(y1, src3, dst3, zero64)
    x1, y2 = _tc_mid(agg1, y1, dis, b1r, W2)
    agg2 = edge_agg(y2, src3, dst3, zero64)
    x2, x_mean = _tc_final(agg2, y2, dis, b2r, batch2)
    return (x2, x_mean, x1, x2)


# speedup vs baseline: 23.2398x; 23.2398x over previous
import functools

import jax
import jax.numpy as jnp
from jax import lax
from jax.experimental import pallas as pl
from jax.experimental.pallas import tpu as pltpu
from jax.experimental.pallas import tpu_sc as plsc

_N = 10000
_E = 320000
_DIN = 128
_H = 64
_G = 64

_NC = 2
_NS = 16
_NW = _NC * _NS
_ET = _E // _NW
_CH = 80
_NCH = _ET // _CH
_ACCR = 10240
_RPT = _ACCR // _NS

_BN = 2000
_GRID = _N // _BN

_built = {}


def _build_sc_kernels():
    mesh = plsc.VectorSubcoreMesh(core_axis_name="c", subcore_axis_name="s")
    cp = pltpu.CompilerParams(use_tc_tiling_on_sc=False)

    @functools.partial(
        pl.kernel,
        mesh=mesh,
        compiler_params=cp,
        out_type=jax.ShapeDtypeStruct((_NC, _ACCR, _H), jnp.float32),
        scratch_types=[
            pltpu.VMEM((_NCH, _CH), jnp.int32),
            pltpu.VMEM((_NCH, _CH), jnp.int32),
            pltpu.VMEM((_CH, _H), jnp.float32),
            pltpu.VMEM_SHARED((_ACCR, _H), jnp.float32),
        ],
    )
    def edge_agg(y_hbm, src_hbm, dst_hbm, zero_hbm, out_hbm, src_v, dst_v,
                 rows_v, acc):
        cid = lax.axis_index("c")
        sid = lax.axis_index("s")
        wid = cid * _NS + sid
        pltpu.sync_copy(zero_hbm, acc.at[pl.ds(sid * _RPT, _RPT)])
        pltpu.sync_copy(src_hbm.at[wid], src_v)
        pltpu.sync_copy(dst_hbm.at[wid], dst_v)
        plsc.subcore_barrier()

        @pl.loop(0, _NCH)
        def _(j):
            pltpu.sync_copy(y_hbm.at[src_v.at[j]], rows_v)
            pltpu.sync_copy(rows_v, acc.at[dst_v.at[j]], add=True)

        plsc.subcore_barrier()
        pltpu.sync_copy(acc.at[pl.ds(sid * _RPT, _RPT)],
                        out_hbm.at[cid, pl.ds(sid * _RPT, _RPT)])

    @functools.partial(
        pl.kernel,
        mesh=mesh,
        compiler_params=cp,
        out_type=jax.ShapeDtypeStruct((_NC, _ACCR, 16), jnp.float32),
        scratch_types=[
            pltpu.VMEM((_NCH, _CH), jnp.int32),
            pltpu.VMEM((_CH, 16), jnp.float32),
            pltpu.VMEM_SHARED((_ACCR, 16), jnp.float32),
        ],
    )
    def deg_hist(dst_hbm, zero_hbm, ones_hbm, out_hbm, dst_v, ones_v, acc):
        cid = lax.axis_index("c")
        sid = lax.axis_index("s")
        wid = cid * _NS + sid
        pltpu.sync_copy(zero_hbm, acc.at[pl.ds(sid * _RPT, _RPT)])
        pltpu.sync_copy(ones_hbm, ones_v)
        pltpu.sync_copy(dst_hbm.at[wid], dst_v)
        plsc.subcore_barrier()

        @pl.loop(0, _NCH)
        def _(j):
            pltpu.sync_copy(ones_v, acc.at[dst_v.at[j]], add=True)

        plsc.subcore_barrier()
        pltpu.sync_copy(acc.at[pl.ds(sid * _RPT, _RPT)],
                        out_hbm.at[cid, pl.ds(sid * _RPT, _RPT)])

    return edge_agg, deg_hist


def _mm1_body(x_ref, w_ref, o_ref):
    o_ref[...] = jnp.dot(x_ref[...], w_ref[...],
                         preferred_element_type=jnp.float32)


def _scale_body(degp_ref, xt_ref, y_ref, dis_ref):
    deg = degp_ref[0, :, :1] + degp_ref[1, :, :1] + 1.0
    dis = lax.rsqrt(deg)
    dis_ref[...] = dis
    y_ref[...] = xt_ref[...] * dis


def _mid_body(agg_ref, y_ref, dis_ref, b_ref, w_ref, x1_ref, y2_ref):
    dis = dis_ref[...]
    s = agg_ref[0] + agg_ref[1] + y_ref[...]
    x1 = jnp.tanh(s * dis + b_ref[...])
    x1_ref[...] = x1
    y2_ref[...] = jnp.dot(x1, w_ref[...],
                          preferred_element_type=jnp.float32) * dis


def _final_body(agg_ref, y_ref, dis_ref, b_ref, batch_ref, x2_ref, mean_ref,
                cnt_ref):
    i = pl.program_id(0)

    @pl.when(i == 0)
    def _():
        mean_ref[...] = jnp.zeros_like(mean_ref)
        cnt_ref[...] = jnp.zeros_like(cnt_ref)

    s = agg_ref[0] + agg_ref[1] + y_ref[...]
    x2 = jnp.tanh(s * dis_ref[...] + b_ref[...])
    x2_ref[...] = x2
    onehot = (batch_ref[...] == lax.broadcasted_iota(
        jnp.int32, (_BN, _G), 1)).astype(jnp.float32)
    mean_ref[...] += lax.dot_general(onehot, x2, (((0,), (0,)), ((), ())),
                                     preferred_element_type=jnp.float32)
    cnt_ref[...] += lax.dot_general(onehot, jnp.ones((_BN, 1), jnp.float32),
                                    (((0,), (0,)), ((), ())),
                                    preferred_element_type=jnp.float32)

    @pl.when(i == _GRID - 1)
    def _():
        mean_ref[...] = mean_ref[...] / jnp.maximum(cnt_ref[...], 1.0)


def _tc_matmul1(attrs, W1):
    return pl.pallas_call(
        _mm1_body,
        grid=(_GRID,),
        in_specs=[
            pl.BlockSpec((_BN, _DIN), lambda i: (i, 0)),
            pl.BlockSpec((_DIN, _H), lambda i: (0, 0)),
        ],
        out_specs=pl.BlockSpec((_BN, _H), lambda i: (i, 0)),
        out_shape=jax.ShapeDtypeStruct((_N, _H), jnp.float32),
    )(attrs, W1)


def _tc_scale(deg_p, xt1):
    return pl.pallas_call(
        _scale_body,
        grid=(_GRID,),
        in_specs=[
            pl.BlockSpec((_NC, _BN, 16), lambda i: (0, i, 0)),
            pl.BlockSpec((_BN, _H), lambda i: (i, 0)),
        ],
        out_specs=[
            pl.BlockSpec((_BN, _H), lambda i: (i, 0)),
            pl.BlockSpec((_BN, 1), lambda i: (i, 0)),
        ],
        out_shape=[
            jax.ShapeDtypeStruct((_N, _H), jnp.float32),
            jax.ShapeDtypeStruct((_N, 1), jnp.float32),
        ],
    )(deg_p, xt1)


def _tc_mid(agg, y1, dis, b1r, W2):
    return pl.pallas_call(
        _mid_body,
        grid=(_GRID,),
        in_specs=[
            pl.BlockSpec((_NC, _BN, _H), lambda i: (0, i, 0)),
            pl.BlockSpec((_BN, _H), lambda i: (i, 0)),
            pl.BlockSpec((_BN, 1), lambda i: (i, 0)),
            pl.BlockSpec((1, _H), lambda i: (0, 0)),
            pl.BlockSpec((_H, _H), lambda i: (0, 0)),
        ],
        out_specs=[
            pl.BlockSpec((_BN, _H), lambda i: (i, 0)),
            pl.BlockSpec((_BN, _H), lambda i: (i, 0)),
        ],
        out_shape=[
            jax.ShapeDtypeStruct((_N, _H), jnp.float32),
            jax.ShapeDtypeStruct((_N, _H), jnp.float32),
        ],
    )(agg, y1, dis, b1r, W2)


def _tc_final(agg, y2, dis, b2r, batch2):
    return pl.pallas_call(
        _final_body,
        grid=(_GRID,),
        in_specs=[
            pl.BlockSpec((_NC, _BN, _H), lambda i: (0, i, 0)),
            pl.BlockSpec((_BN, _H), lambda i: (i, 0)),
            pl.BlockSpec((_BN, 1), lambda i: (i, 0)),
            pl.BlockSpec((1, _H), lambda i: (0, 0)),
            pl.BlockSpec((_BN, 1), lambda i: (i, 0)),
        ],
        out_specs=[
            pl.BlockSpec((_BN, _H), lambda i: (i, 0)),
            pl.BlockSpec((_G, _H), lambda i: (0, 0)),
        ],
        out_shape=[
            jax.ShapeDtypeStruct((_N, _H), jnp.float32),
            jax.ShapeDtypeStruct((_G, _H), jnp.float32),
        ],
        scratch_shapes=[pltpu.VMEM((_G, 1), jnp.float32)],
    )(agg, y2, dis, b2r, batch2)


def kernel(attrs, edge_index, batch, W1, b1, W2, b2):
    if "sc" not in _built:
        _built["sc"] = _build_sc_kernels()
    edge_agg, deg_hist = _built["sc"]

    f32 = jnp.float32
    src3 = edge_index[0].reshape(_NW, _NCH, _CH)
    dst3 = edge_index[1].reshape(_NW, _NCH, _CH)
    zero64 = jnp.zeros((_RPT, _H), f32)
    zero16 = jnp.zeros((_RPT, 16), f32)
    ones16 = jnp.ones((_CH, 16), f32)
    b1r = b1.reshape(1, _H)
    b2r = b2.reshape(1, _H)
    batch2 = batch.reshape(_N, 1)

    deg_p = deg_hist(dst3, zero16, ones16)
    xt1 = _tc_matmul1(attrs.astype(f32), W1)
    y1, dis = _tc_scale(deg_p, xt1)
    agg1 = edge_agg(y1, src3, dst3, zero64)
    x1, y2 = _tc_mid(agg1, y1, dis, b1r, W2)
    agg2 = edge_agg(y2, src3, dst3, zero64)
    x2, x_mean = _tc_final(agg2, y2, dis, b2r, batch2)
    return (x2, x_mean, x1, x2)


# speedup vs baseline: 26.9349x; 1.1590x over previous
import functools

import jax
import jax.numpy as jnp
from jax import lax
from jax.experimental import pallas as pl
from jax.experimental.pallas import tpu as pltpu
from jax.experimental.pallas import tpu_sc as plsc

_N = 10000
_E = 320000
_DIN = 128
_H = 64
_G = 64

_NC = 2
_NS = 16
_NW = _NC * _NS
_ET = _E // _NW
_CH = 125
_NCH = _ET // _CH
_ACCR = 10240
_RPT = _ACCR // _NS

_BN = 2000
_GRID = _N // _BN

_built = {}


def _build_sc_kernels():
    mesh = plsc.VectorSubcoreMesh(core_axis_name="c", subcore_axis_name="s")
    cp = pltpu.CompilerParams(use_tc_tiling_on_sc=False)

    @functools.partial(
        pl.kernel,
        mesh=mesh,
        compiler_params=cp,
        out_type=jax.ShapeDtypeStruct((_NC, _ACCR, _H), jnp.float32),
        scratch_types=[
            pltpu.VMEM((_NCH, _CH), jnp.int32),
            pltpu.VMEM((_NCH, _CH), jnp.int32),
            pltpu.VMEM((_CH, _H), jnp.float32),
            pltpu.VMEM_SHARED((_ACCR, _H), jnp.float32),
        ],
    )
    def edge_agg(y_hbm, src_hbm, dst_hbm, zero_hbm, out_hbm, src_v, dst_v,
                 rows_v, acc):
        cid = lax.axis_index("c")
        sid = lax.axis_index("s")
        wid = cid * _NS + sid
        pltpu.sync_copy(zero_hbm, acc.at[pl.ds(sid * _RPT, _RPT)])
        pltpu.sync_copy(src_hbm.at[wid], src_v)
        pltpu.sync_copy(dst_hbm.at[wid], dst_v)
        plsc.subcore_barrier()

        @pl.loop(0, _NCH)
        def _(j):
            pltpu.sync_copy(y_hbm.at[src_v.at[j]], rows_v)
            pltpu.sync_copy(rows_v, acc.at[dst_v.at[j]], add=True)

        plsc.subcore_barrier()
        pltpu.sync_copy(acc.at[pl.ds(sid * _RPT, _RPT)],
                        out_hbm.at[cid, pl.ds(sid * _RPT, _RPT)])

    @functools.partial(
        pl.kernel,
        mesh=mesh,
        compiler_params=cp,
        out_type=jax.ShapeDtypeStruct((_NC, _ACCR, 16), jnp.float32),
        scratch_types=[
            pltpu.VMEM((_NCH, _CH), jnp.int32),
            pltpu.VMEM((_CH, 16), jnp.float32),
            pltpu.VMEM_SHARED((_ACCR, 16), jnp.float32),
        ],
    )
    def deg_hist(dst_hbm, zero_hbm, ones_hbm, out_hbm, dst_v, ones_v, acc):
        cid = lax.axis_index("c")
        sid = lax.axis_index("s")
        wid = cid * _NS + sid
        pltpu.sync_copy(zero_hbm, acc.at[pl.ds(sid * _RPT, _RPT)])
        pltpu.sync_copy(ones_hbm, ones_v)
        pltpu.sync_copy(dst_hbm.at[wid], dst_v)
        plsc.subcore_barrier()

        @pl.loop(0, _NCH)
        def _(j):
            pltpu.sync_copy(ones_v, acc.at[dst_v.at[j]], add=True)

        plsc.subcore_barrier()
        pltpu.sync_copy(acc.at[pl.ds(sid * _RPT, _RPT)],
                        out_hbm.at[cid, pl.ds(sid * _RPT, _RPT)])

    return edge_agg, deg_hist


def _mm1_body(x_ref, w_ref, o_ref):
    o_ref[...] = jnp.dot(x_ref[...], w_ref[...],
                         preferred_element_type=jnp.float32)


def _scale_body(degp_ref, xt_ref, y_ref, dis_ref):
    deg = degp_ref[0, :, :1] + degp_ref[1, :, :1] + 1.0
    dis = lax.rsqrt(deg)
    dis_ref[...] = dis
    y_ref[...] = xt_ref[...] * dis


def _mid_body(agg_ref, y_ref, dis_ref, b_ref, w_ref, x1_ref, y2_ref):
    dis = dis_ref[...]
    s = agg_ref[0] + agg_ref[1] + y_ref[...]
    x1 = jnp.tanh(s * dis + b_ref[...])
    x1_ref[...] = x1
    y2_ref[...] = jnp.dot(x1, w_ref[...],
                          preferred_element_type=jnp.float32) * dis


def _final_body(agg_ref, y_ref, dis_ref, b_ref, batch_ref, x2_ref, mean_ref,
                cnt_ref):
    i = pl.program_id(0)

    @pl.when(i == 0)
    def _():
        mean_ref[...] = jnp.zeros_like(mean_ref)
        cnt_ref[...] = jnp.zeros_like(cnt_ref)

    s = agg_ref[0] + agg_ref[1] + y_ref[...]
    x2 = jnp.tanh(s * dis_ref[...] + b_ref[...])
    x2_ref[...] = x2
    onehot = (batch_ref[...] == lax.broadcasted_iota(
        jnp.int32, (_BN, _G), 1)).astype(jnp.float32)
    mean_ref[...] += lax.dot_general(onehot, x2, (((0,), (0,)), ((), ())),
                                     preferred_element_type=jnp.float32)
    cnt_ref[...] += lax.dot_general(onehot, jnp.ones((_BN, 1), jnp.float32),
                                    (((0,), (0,)), ((), ())),
                                    preferred_element_type=jnp.float32)

    @pl.when(i == _GRID - 1)
    def _():
        mean_ref[...] = mean_ref[...] / jnp.maximum(cnt_ref[...], 1.0)


def _tc_matmul1(attrs, W1):
    return pl.pallas_call(
        _mm1_body,
        grid=(_GRID,),
        in_specs=[
            pl.BlockSpec((_BN, _DIN), lambda i: (i, 0)),
            pl.BlockSpec((_DIN, _H), lambda i: (0, 0)),
        ],
        out_specs=pl.BlockSpec((_BN, _H), lambda i: (i, 0)),
        out_shape=jax.ShapeDtypeStruct((_N, _H), jnp.float32),
    )(attrs, W1)


def _tc_scale(deg_p, xt1):
    return pl.pallas_call(
        _scale_body,
        grid=(_GRID,),
        in_specs=[
            pl.BlockSpec((_NC, _BN, 16), lambda i: (0, i, 0)),
            pl.BlockSpec((_BN, _H), lambda i: (i, 0)),
        ],
        out_specs=[
            pl.BlockSpec((_BN, _H), lambda i: (i, 0)),
            pl.BlockSpec((_BN, 1), lambda i: (i, 0)),
        ],
        out_shape=[
            jax.ShapeDtypeStruct((_N, _H), jnp.float32),
            jax.ShapeDtypeStruct((_N, 1), jnp.float32),
        ],
    )(deg_p, xt1)


def _tc_mid(agg, y1, dis, b1r, W2):
    return pl.pallas_call(
        _mid_body,
        grid=(_GRID,),
        in_specs=[
            pl.BlockSpec((_NC, _BN, _H), lambda i: (0, i, 0)),
            pl.BlockSpec((_BN, _H), lambda i: (i, 0)),
            pl.BlockSpec((_BN, 1), lambda i: (i, 0)),
            pl.BlockSpec((1, _H), lambda i: (0, 0)),
            pl.BlockSpec((_H, _H), lambda i: (0, 0)),
        ],
        out_specs=[
            pl.BlockSpec((_BN, _H), lambda i: (i, 0)),
            pl.BlockSpec((_BN, _H), lambda i: (i, 0)),
        ],
        out_shape=[
            jax.ShapeDtypeStruct((_N, _H), jnp.float32),
            jax.ShapeDtypeStruct((_N, _H), jnp.float32),
        ],
    )(agg, y1, dis, b1r, W2)


def _tc_final(agg, y2, dis, b2r, batch2):
    return pl.pallas_call(
        _final_body,
        grid=(_GRID,),
        in_specs=[
            pl.BlockSpec((_NC, _BN, _H), lambda i: (0, i, 0)),
            pl.BlockSpec((_BN, _H), lambda i: (i, 0)),
            pl.BlockSpec((_BN, 1), lambda i: (i, 0)),
            pl.BlockSpec((1, _H), lambda i: (0, 0)),
            pl.BlockSpec((_BN, 1), lambda i: (i, 0)),
        ],
        out_specs=[
            pl.BlockSpec((_BN, _H), lambda i: (i, 0)),
            pl.BlockSpec((_G, _H), lambda i: (0, 0)),
        ],
        out_shape=[
            jax.ShapeDtypeStruct((_N, _H), jnp.float32),
            jax.ShapeDtypeStruct((_G, _H), jnp.float32),
        ],
        scratch_shapes=[pltpu.VMEM((_G, 1), jnp.float32)],
    )(agg, y2, dis, b2r, batch2)


def kernel(attrs, edge_index, batch, W1, b1, W2, b2):
    if "sc" not in _built:
        _built["sc"] = _build_sc_kernels()
    edge_agg, deg_hist = _built["sc"]

    f32 = jnp.float32
    src3 = edge_index[0].reshape(_NW, _NCH, _CH)
    dst3 = edge_index[1].reshape(_NW, _NCH, _CH)
    zero64 = jnp.zeros((_RPT, _H), f32)
    zero16 = jnp.zeros((_RPT, 16), f32)
    ones16 = jnp.ones((_CH, 16), f32)
    b1r = b1.reshape(1, _H)
    b2r = b2.reshape(1, _H)
    batch2 = batch.reshape(_N, 1)

    deg_p = deg_hist(dst3, zero16, ones16)
    xt1 = _tc_matmul1(attrs.astype(f32), W1)
    y1, dis = _tc_scale(deg_p, xt1)
    agg1 = edge_agg(y1, src3, dst3, zero64)
    x1, y2 = _tc_mid(agg1, y1, dis, b1r, W2)
    agg2 = edge_agg(y2, src3, dst3, zero64)
    x2, x_mean = _tc_final(agg2, y2, dis, b2r, batch2)
    return (x2, x_mean, x1, x2)


# speedup vs baseline: 34.2862x; 1.2729x over previous
import functools

import jax
import jax.numpy as jnp
from jax import lax
from jax.experimental import pallas as pl
from jax.experimental.pallas import tpu as pltpu
from jax.experimental.pallas import tpu_sc as plsc

_N = 10000
_E = 320000
_DIN = 128
_H = 64
_G = 64

_NC = 2
_NS = 16
_NW = _NC * _NS
_ET = _E // _NW
_CH = 125
_NCH = _ET // _CH
_NB = 4
_NR = _NCH // _NB
_ACCR = 10240
_RPT = _ACCR // _NS

_BN = 2000
_GRID = _N // _BN

_built = {}


def _build_sc_kernels():
    mesh = plsc.VectorSubcoreMesh(core_axis_name="c", subcore_axis_name="s")
    cp = pltpu.CompilerParams(use_tc_tiling_on_sc=False)

    @functools.partial(
        pl.kernel,
        mesh=mesh,
        compiler_params=cp,
        out_type=jax.ShapeDtypeStruct((_NC, _ACCR, _H), jnp.float32),
        scratch_types=[
            pltpu.VMEM((_NCH, _CH), jnp.int32),
            pltpu.VMEM((_NCH, _CH), jnp.int32),
            pltpu.VMEM((_NB, _CH, _H), jnp.float32),
            pltpu.VMEM_SHARED((_ACCR, _H), jnp.float32),
        ] + [pltpu.SemaphoreType.DMA] * (2 * _NB),
    )
    def edge_agg(y_hbm, src_hbm, dst_hbm, zero_hbm, out_hbm, src_v, dst_v,
                 rows_v, acc, *sems):
        gsem = sems[:_NB]
        ssem = sems[_NB:]
        cid = lax.axis_index("c")
        sid = lax.axis_index("s")
        wid = cid * _NS + sid
        pltpu.sync_copy(zero_hbm, acc.at[pl.ds(sid * _RPT, _RPT)])
        pltpu.sync_copy(src_hbm.at[wid], src_v)
        pltpu.sync_copy(dst_hbm.at[wid], dst_v)
        plsc.subcore_barrier()

        @pl.loop(0, _NR)
        def _(i):
            j0 = i * _NB
            gh = [pltpu.async_copy(y_hbm.at[src_v.at[j0 + b]], rows_v.at[b],
                                   gsem[b])
                  for b in range(_NB)]
            sh = []
            for b in range(_NB):
                gh[b].wait()
                sh.append(pltpu.async_copy(rows_v.at[b],
                                           acc.at[dst_v.at[j0 + b]],
                                           ssem[b], add=True))
            for h in sh:
                h.wait()

        plsc.subcore_barrier()
        pltpu.sync_copy(acc.at[pl.ds(sid * _RPT, _RPT)],
                        out_hbm.at[cid, pl.ds(sid * _RPT, _RPT)])

    @functools.partial(
        pl.kernel,
        mesh=mesh,
        compiler_params=cp,
        out_type=jax.ShapeDtypeStruct((_NC, _ACCR, 16), jnp.float32),
        scratch_types=[
            pltpu.VMEM((_NCH, _CH), jnp.int32),
            pltpu.VMEM((_CH, 16), jnp.float32),
            pltpu.VMEM_SHARED((_ACCR, 16), jnp.float32),
        ],
    )
    def deg_hist(dst_hbm, zero_hbm, ones_hbm, out_hbm, dst_v, ones_v, acc):
        cid = lax.axis_index("c")
        sid = lax.axis_index("s")
        wid = cid * _NS + sid
        pltpu.sync_copy(zero_hbm, acc.at[pl.ds(sid * _RPT, _RPT)])
        pltpu.sync_copy(ones_hbm, ones_v)
        pltpu.sync_copy(dst_hbm.at[wid], dst_v)
        plsc.subcore_barrier()

        @pl.loop(0, _NCH)
        def _(j):
            pltpu.sync_copy(ones_v, acc.at[dst_v.at[j]], add=True)

        plsc.subcore_barrier()
        pltpu.sync_copy(acc.at[pl.ds(sid * _RPT, _RPT)],
                        out_hbm.at[cid, pl.ds(sid * _RPT, _RPT)])

    return edge_agg, deg_hist


def _mm1_body(x_ref, w_ref, o_ref):
    o_ref[...] = jnp.dot(x_ref[...], w_ref[...],
                         preferred_element_type=jnp.float32)


def _scale_body(degp_ref, xt_ref, y_ref, dis_ref):
    deg = degp_ref[0, :, :1] + degp_ref[1, :, :1] + 1.0
    dis = lax.rsqrt(deg)
    dis_ref[...] = dis
    y_ref[...] = xt_ref[...] * dis


def _mid_body(agg_ref, y_ref, dis_ref, b_ref, w_ref, x1_ref, y2_ref):
    dis = dis_ref[...]
    s = agg_ref[0] + agg_ref[1] + y_ref[...]
    x1 = jnp.tanh(s * dis + b_ref[...])
    x1_ref[...] = x1
    y2_ref[...] = jnp.dot(x1, w_ref[...],
                          preferred_element_type=jnp.float32) * dis


def _final_body(agg_ref, y_ref, dis_ref, b_ref, batch_ref, x2_ref, mean_ref,
                cnt_ref):
    i = pl.program_id(0)

    @pl.when(i == 0)
    def _():
        mean_ref[...] = jnp.zeros_like(mean_ref)
        cnt_ref[...] = jnp.zeros_like(cnt_ref)

    s = agg_ref[0] + agg_ref[1] + y_ref[...]
    x2 = jnp.tanh(s * dis_ref[...] + b_ref[...])
    x2_ref[...] = x2
    onehot = (batch_ref[...] == lax.broadcasted_iota(
        jnp.int32, (_BN, _G), 1)).astype(jnp.float32)
    mean_ref[...] += lax.dot_general(onehot, x2, (((0,), (0,)), ((), ())),
                                     preferred_element_type=jnp.float32)
    cnt_ref[...] += lax.dot_general(onehot, jnp.ones((_BN, 1), jnp.float32),
                                    (((0,), (0,)), ((), ())),
                                    preferred_element_type=jnp.float32)

    @pl.when(i == _GRID - 1)
    def _():
        mean_ref[...] = mean_ref[...] / jnp.maximum(cnt_ref[...], 1.0)


def _tc_matmul1(attrs, W1):
    return pl.pallas_call(
        _mm1_body,
        grid=(_GRID,),
        in_specs=[
            pl.BlockSpec((_BN, _DIN), lambda i: (i, 0)),
            pl.BlockSpec((_DIN, _H), lambda i: (0, 0)),
        ],
        out_specs=pl.BlockSpec((_BN, _H), lambda i: (i, 0)),
        out_shape=jax.ShapeDtypeStruct((_N, _H), jnp.float32),
    )(attrs, W1)


def _tc_scale(deg_p, xt1):
    return pl.pallas_call(
        _scale_body,
        grid=(_GRID,),
        in_specs=[
            pl.BlockSpec((_NC, _BN, 16), lambda i: (0, i, 0)),
            pl.BlockSpec((_BN, _H), lambda i: (i, 0)),
        ],
        out_specs=[
            pl.BlockSpec((_BN, _H), lambda i: (i, 0)),
            pl.BlockSpec((_BN, 1), lambda i: (i, 0)),
        ],
        out_shape=[
            jax.ShapeDtypeStruct((_N, _H), jnp.float32),
            jax.ShapeDtypeStruct((_N, 1), jnp.float32),
        ],
    )(deg_p, xt1)


def _tc_mid(agg, y1, dis, b1r, W2):
    return pl.pallas_call(
        _mid_body,
        grid=(_GRID,),
        in_specs=[
            pl.BlockSpec((_NC, _BN, _H), lambda i: (0, i, 0)),
            pl.BlockSpec((_BN, _H), lambda i: (i, 0)),
            pl.BlockSpec((_BN, 1), lambda i: (i, 0)),
            pl.BlockSpec((1, _H), lambda i: (0, 0)),
            pl.BlockSpec((_H, _H), lambda i: (0, 0)),
        ],
        out_specs=[
            pl.BlockSpec((_BN, _H), lambda i: (i, 0)),
            pl.BlockSpec((_BN, _H), lambda i: (i, 0)),
        ],
        out_shape=[
            jax.ShapeDtypeStruct((_N, _H), jnp.float32),
            jax.ShapeDtypeStruct((_N, _H), jnp.float32),
        ],
    )(agg, y1, dis, b1r, W2)


def _tc_final(agg, y2, dis, b2r, batch2):
    return pl.pallas_call(
        _final_body,
        grid=(_GRID,),
        in_specs=[
            pl.BlockSpec((_NC, _BN, _H), lambda i: (0, i, 0)),
            pl.BlockSpec((_BN, _H), lambda i: (i, 0)),
            pl.BlockSpec((_BN, 1), lambda i: (i, 0)),
            pl.BlockSpec((1, _H), lambda i: (0, 0)),
            pl.BlockSpec((_BN, 1), lambda i: (i, 0)),
        ],
        out_specs=[
            pl.BlockSpec((_BN, _H), lambda i: (i, 0)),
            pl.BlockSpec((_G, _H), lambda i: (0, 0)),
        ],
        out_shape=[
            jax.ShapeDtypeStruct((_N, _H), jnp.float32),
            jax.ShapeDtypeStruct((_G, _H), jnp.float32),
        ],
        scratch_shapes=[pltpu.VMEM((_G, 1), jnp.float32)],
    )(agg, y2, dis, b2r, batch2)


def kernel(attrs, edge_index, batch, W1, b1, W2, b2):
    if "sc" not in _built:
        _built["sc"] = _build_sc_kernels()
    edge_agg, deg_hist = _built["sc"]

    f32 = jnp.float32
    src3 = edge_index[0].reshape(_NW, _NCH, _CH)
    dst3 = edge_index[1].reshape(_NW, _NCH, _CH)
    zero64 = jnp.zeros((_RPT, _H), f32)
    zero16 = jnp.zeros((_RPT, 16), f32)
    ones16 = jnp.ones((_CH, 16), f32)
    b1r = b1.reshape(1, _H)
    b2r = b2.reshape(1, _H)
    batch2 = batch.reshape(_N, 1)

    deg_p = deg_hist(dst3, zero16, ones16)
    xt1 = _tc_matmul1(attrs.astype(f32), W1)
    y1, dis = _tc_scale(deg_p, xt1)
    agg1 = edge_agg(y1, src3, dst3, zero64)
    x1, y2 = _tc_mid(agg1, y1, dis, b1r, W2)
    agg2 = edge_agg(y2, src3, dst3, zero64)
    x2, x_mean = _tc_final(agg2, y2, dis, b2r, batch2)
    return (x2, x_mean, x1, x2)


# speedup vs baseline: 36.1850x; 1.0554x over previous
import functools

import jax
import jax.numpy as jnp
from jax import lax
from jax.experimental import pallas as pl
from jax.experimental.pallas import tpu as pltpu
from jax.experimental.pallas import tpu_sc as plsc

_N = 10000
_E = 320000
_DIN = 128
_H = 64
_G = 64

_NC = 2
_NS = 16
_NW = _NC * _NS
_ET = _E // _NW
_CH = 125
_NCH = _ET // _CH
_NB = 8
_NR = _NCH // _NB
_ACCR = 10240
_RPT = _ACCR // _NS

_BN = 2000
_GRID = _N // _BN

_built = {}


def _build_sc_kernels():
    mesh = plsc.VectorSubcoreMesh(core_axis_name="c", subcore_axis_name="s")
    cp = pltpu.CompilerParams(use_tc_tiling_on_sc=False)

    @functools.partial(
        pl.kernel,
        mesh=mesh,
        compiler_params=cp,
        out_type=jax.ShapeDtypeStruct((_NC, _ACCR, _H), jnp.float32),
        scratch_types=[
            pltpu.VMEM((_NCH, _CH), jnp.int32),
            pltpu.VMEM((_NCH, _CH), jnp.int32),
            pltpu.VMEM((_NB, _CH, _H), jnp.float32),
            pltpu.VMEM_SHARED((_ACCR, _H), jnp.float32),
        ] + [pltpu.SemaphoreType.DMA] * (2 * _NB),
    )
    def edge_agg(y_hbm, src_hbm, dst_hbm, zero_hbm, out_hbm, src_v, dst_v,
                 rows_v, acc, *sems):
        gsem = sems[:_NB]
        ssem = sems[_NB:]
        cid = lax.axis_index("c")
        sid = lax.axis_index("s")
        wid = cid * _NS + sid
        pltpu.sync_copy(zero_hbm, acc.at[pl.ds(sid * _RPT, _RPT)])
        pltpu.sync_copy(src_hbm.at[wid], src_v)
        pltpu.sync_copy(dst_hbm.at[wid], dst_v)
        plsc.subcore_barrier()

        @pl.loop(0, _NR)
        def _(i):
            j0 = i * _NB
            gh = [pltpu.async_copy(y_hbm.at[src_v.at[j0 + b]], rows_v.at[b],
                                   gsem[b])
                  for b in range(_NB)]
            sh = []
            for b in range(_NB):
                gh[b].wait()
                sh.append(pltpu.async_copy(rows_v.at[b],
                                           acc.at[dst_v.at[j0 + b]],
                                           ssem[b], add=True))
            for h in sh:
                h.wait()

        plsc.subcore_barrier()
        pltpu.sync_copy(acc.at[pl.ds(sid * _RPT, _RPT)],
                        out_hbm.at[cid, pl.ds(sid * _RPT, _RPT)])

    @functools.partial(
        pl.kernel,
        mesh=mesh,
        compiler_params=cp,
        out_type=jax.ShapeDtypeStruct((_NC, _ACCR, 16), jnp.float32),
        scratch_types=[
            pltpu.VMEM((_NCH, _CH), jnp.int32),
            pltpu.VMEM((_CH, 16), jnp.float32),
            pltpu.VMEM_SHARED((_ACCR, 16), jnp.float32),
        ],
    )
    def deg_hist(dst_hbm, zero_hbm, ones_hbm, out_hbm, dst_v, ones_v, acc):
        cid = lax.axis_index("c")
        sid = lax.axis_index("s")
        wid = cid * _NS + sid
        pltpu.sync_copy(zero_hbm, acc.at[pl.ds(sid * _RPT, _RPT)])
        pltpu.sync_copy(ones_hbm, ones_v)
        pltpu.sync_copy(dst_hbm.at[wid], dst_v)
        plsc.subcore_barrier()

        @pl.loop(0, _NCH)
        def _(j):
            pltpu.sync_copy(ones_v, acc.at[dst_v.at[j]], add=True)

        plsc.subcore_barrier()
        pltpu.sync_copy(acc.at[pl.ds(sid * _RPT, _RPT)],
                        out_hbm.at[cid, pl.ds(sid * _RPT, _RPT)])

    return edge_agg, deg_hist


def _mm1_body(x_ref, w_ref, o_ref):
    o_ref[...] = jnp.dot(x_ref[...], w_ref[...],
                         preferred_element_type=jnp.float32)


def _scale_body(degp_ref, xt_ref, y_ref, dis_ref):
    deg = degp_ref[0, :, :1] + degp_ref[1, :, :1] + 1.0
    dis = lax.rsqrt(deg)
    dis_ref[...] = dis
    y_ref[...] = xt_ref[...] * dis


def _mid_body(agg_ref, y_ref, dis_ref, b_ref, w_ref, x1_ref, y2_ref):
    dis = dis_ref[...]
    s = agg_ref[0] + agg_ref[1] + y_ref[...]
    x1 = jnp.tanh(s * dis + b_ref[...])
    x1_ref[...] = x1
    y2_ref[...] = jnp.dot(x1, w_ref[...],
                          preferred_element_type=jnp.float32) * dis


def _final_body(agg_ref, y_ref, dis_ref, b_ref, batch_ref, x2_ref, mean_ref,
                cnt_ref):
    i = pl.program_id(0)

    @pl.when(i == 0)
    def _():
        mean_ref[...] = jnp.zeros_like(mean_ref)
        cnt_ref[...] = jnp.zeros_like(cnt_ref)

    s = agg_ref[0] + agg_ref[1] + y_ref[...]
    x2 = jnp.tanh(s * dis_ref[...] + b_ref[...])
    x2_ref[...] = x2
    onehot = (batch_ref[...] == lax.broadcasted_iota(
        jnp.int32, (_BN, _G), 1)).astype(jnp.float32)
    mean_ref[...] += lax.dot_general(onehot, x2, (((0,), (0,)), ((), ())),
                                     preferred_element_type=jnp.float32)
    cnt_ref[...] += lax.dot_general(onehot, jnp.ones((_BN, 1), jnp.float32),
                                    (((0,), (0,)), ((), ())),
                                    preferred_element_type=jnp.float32)

    @pl.when(i == _GRID - 1)
    def _():
        mean_ref[...] = mean_ref[...] / jnp.maximum(cnt_ref[...], 1.0)


def _tc_matmul1(attrs, W1):
    return pl.pallas_call(
        _mm1_body,
        grid=(_GRID,),
        in_specs=[
            pl.BlockSpec((_BN, _DIN), lambda i: (i, 0)),
            pl.BlockSpec((_DIN, _H), lambda i: (0, 0)),
        ],
        out_specs=pl.BlockSpec((_BN, _H), lambda i: (i, 0)),
        out_shape=jax.ShapeDtypeStruct((_N, _H), jnp.float32),
    )(attrs, W1)


def _tc_scale(deg_p, xt1):
    return pl.pallas_call(
        _scale_body,
        grid=(_GRID,),
        in_specs=[
            pl.BlockSpec((_NC, _BN, 16), lambda i: (0, i, 0)),
            pl.BlockSpec((_BN, _H), lambda i: (i, 0)),
        ],
        out_specs=[
            pl.BlockSpec((_BN, _H), lambda i: (i, 0)),
            pl.BlockSpec((_BN, 1), lambda i: (i, 0)),
        ],
        out_shape=[
            jax.ShapeDtypeStruct((_N, _H), jnp.float32),
            jax.ShapeDtypeStruct((_N, 1), jnp.float32),
        ],
    )(deg_p, xt1)


def _tc_mid(agg, y1, dis, b1r, W2):
    return pl.pallas_call(
        _mid_body,
        grid=(_GRID,),
        in_specs=[
            pl.BlockSpec((_NC, _BN, _H), lambda i: (0, i, 0)),
            pl.BlockSpec((_BN, _H), lambda i: (i, 0)),
            pl.BlockSpec((_BN, 1), lambda i: (i, 0)),
            pl.BlockSpec((1, _H), lambda i: (0, 0)),
            pl.BlockSpec((_H, _H), lambda i: (0, 0)),
        ],
        out_specs=[
            pl.BlockSpec((_BN, _H), lambda i: (i, 0)),
            pl.BlockSpec((_BN, _H), lambda i: (i, 0)),
        ],
        out_shape=[
            jax.ShapeDtypeStruct((_N, _H), jnp.float32),
            jax.ShapeDtypeStruct((_N, _H), jnp.float32),
        ],
    )(agg, y1, dis, b1r, W2)


def _tc_final(agg, y2, dis, b2r, batch2):
    return pl.pallas_call(
        _final_body,
        grid=(_GRID,),
        in_specs=[
            pl.BlockSpec((_NC, _BN, _H), lambda i: (0, i, 0)),
            pl.BlockSpec((_BN, _H), lambda i: (i, 0)),
            pl.BlockSpec((_BN, 1), lambda i: (i, 0)),
            pl.BlockSpec((1, _H), lambda i: (0, 0)),
            pl.BlockSpec((_BN, 1), lambda i: (i, 0)),
        ],
        out_specs=[
            pl.BlockSpec((_BN, _H), lambda i: (i, 0)),
            pl.BlockSpec((_G, _H), lambda i: (0, 0)),
        ],
        out_shape=[
            jax.ShapeDtypeStruct((_N, _H), jnp.float32),
            jax.ShapeDtypeStruct((_G, _H), jnp.float32),
        ],
        scratch_shapes=[pltpu.VMEM((_G, 1), jnp.float32)],
    )(agg, y2, dis, b2r, batch2)


def kernel(attrs, edge_index, batch, W1, b1, W2, b2):
    if "sc" not in _built:
        _built["sc"] = _build_sc_kernels()
    edge_agg, deg_hist = _built["sc"]

    f32 = jnp.float32
    src3 = edge_index[0].reshape(_NW, _NCH, _CH)
    dst3 = edge_index[1].reshape(_NW, _NCH, _CH)
    zero64 = jnp.zeros((_RPT, _H), f32)
    zero16 = jnp.zeros((_RPT, 16), f32)
    ones16 = jnp.ones((_CH, 16), f32)
    b1r = b1.reshape(1, _H)
    b2r = b2.reshape(1, _H)
    batch2 = batch.reshape(_N, 1)

    deg_p = deg_hist(dst3, zero16, ones16)
    xt1 = _tc_matmul1(attrs.astype(f32), W1)
    y1, dis = _tc_scale(deg_p, xt1)
    agg1 = edge_agg(y1, src3, dst3, zero64)
    x1, y2 = _tc_mid(agg1, y1, dis, b1r, W2)
    agg2 = edge_agg(y2, src3, dst3, zero64)
    x2, x_mean = _tc_final(agg2, y2, dis, b2r, batch2)
    return (x2, x_mean, x1, x2)


# speedup vs baseline: 39.5930x; 1.0942x over previous
import functools

import jax
import jax.numpy as jnp
from jax import lax
from jax.experimental import pallas as pl
from jax.experimental.pallas import tpu as pltpu
from jax.experimental.pallas import tpu_sc as plsc

_N = 10000
_E = 320000
_DIN = 128
_H = 64
_G = 64

_NC = 2
_NS = 16
_NW = _NC * _NS
_ET = _E // _NW
_CH = 125
_NCH = _ET // _CH
_NB = 8
_NR = _NCH // _NB
_ACCR = 10240
_RPT = _ACCR // _NS

_BN = 2000
_GRID = _N // _BN

_built = {}


def _build_sc_kernels():
    mesh = plsc.VectorSubcoreMesh(core_axis_name="c", subcore_axis_name="s")
    cp = pltpu.CompilerParams(use_tc_tiling_on_sc=False)

    @functools.partial(
        pl.kernel,
        mesh=mesh,
        compiler_params=cp,
        out_type=jax.ShapeDtypeStruct((_NC, _ACCR, _H), jnp.float32),
        scratch_types=[
            pltpu.VMEM((_NCH, _CH), jnp.int32),
            pltpu.VMEM((_NCH, _CH), jnp.int32),
            pltpu.VMEM((_NB, _CH, _H), jnp.float32),
            pltpu.VMEM_SHARED((_ACCR, _H), jnp.float32),
        ] + [pltpu.SemaphoreType.DMA] * (2 * _NB),
    )
    def edge_agg(y_hbm, src_hbm, dst_hbm, zero_hbm, out_hbm, src_v, dst_v,
                 rows_v, acc, *sems):
        gsem = sems[:_NB]
        ssem = sems[_NB:]
        cid = lax.axis_index("c")
        sid = lax.axis_index("s")
        wid = cid * _NS + sid
        pltpu.sync_copy(zero_hbm, acc.at[pl.ds(sid * _RPT, _RPT)])
        pltpu.sync_copy(src_hbm.at[wid], src_v)
        pltpu.sync_copy(dst_hbm.at[wid], dst_v)
        plsc.subcore_barrier()

        for b in range(_NB):
            pltpu.async_copy(y_hbm.at[src_v.at[b]], rows_v.at[b], gsem[b])

        @pl.loop(0, _NR - 1)
        def _(i):
            j0 = i * _NB
            sh = []
            for b in range(_NB):
                pltpu.make_async_copy(y_hbm.at[src_v.at[j0 + b]],
                                      rows_v.at[b], gsem[b]).wait()
                sh.append(pltpu.async_copy(rows_v.at[b],
                                           acc.at[dst_v.at[j0 + b]],
                                           ssem[b], add=True))
            for b in range(_NB):
                sh[b].wait()
                pltpu.async_copy(y_hbm.at[src_v.at[j0 + _NB + b]],
                                 rows_v.at[b], gsem[b])

        jL = (_NR - 1) * _NB
        shl = []
        for b in range(_NB):
            pltpu.make_async_copy(y_hbm.at[src_v.at[jL + b]],
                                  rows_v.at[b], gsem[b]).wait()
            shl.append(pltpu.async_copy(rows_v.at[b],
                                        acc.at[dst_v.at[jL + b]],
                                        ssem[b], add=True))
        for h in shl:
            h.wait()

        plsc.subcore_barrier()
        pltpu.sync_copy(acc.at[pl.ds(sid * _RPT, _RPT)],
                        out_hbm.at[cid, pl.ds(sid * _RPT, _RPT)])

    @functools.partial(
        pl.kernel,
        mesh=mesh,
        compiler_params=cp,
        out_type=jax.ShapeDtypeStruct((_NC, _ACCR, 16), jnp.float32),
        scratch_types=[
            pltpu.VMEM((_NCH, _CH), jnp.int32),
            pltpu.VMEM((_CH, 16), jnp.float32),
            pltpu.VMEM_SHARED((_ACCR, 16), jnp.float32),
        ],
    )
    def deg_hist(dst_hbm, zero_hbm, ones_hbm, out_hbm, dst_v, ones_v, acc):
        cid = lax.axis_index("c")
        sid = lax.axis_index("s")
        wid = cid * _NS + sid
        pltpu.sync_copy(zero_hbm, acc.at[pl.ds(sid * _RPT, _RPT)])
        pltpu.sync_copy(ones_hbm, ones_v)
        pltpu.sync_copy(dst_hbm.at[wid], dst_v)
        plsc.subcore_barrier()

        @pl.loop(0, _NCH)
        def _(j):
            pltpu.sync_copy(ones_v, acc.at[dst_v.at[j]], add=True)

        plsc.subcore_barrier()
        pltpu.sync_copy(acc.at[pl.ds(sid * _RPT, _RPT)],
                        out_hbm.at[cid, pl.ds(sid * _RPT, _RPT)])

    return edge_agg, deg_hist


def _mm1_body(x_ref, w_ref, o_ref):
    o_ref[...] = jnp.dot(x_ref[...], w_ref[...],
                         preferred_element_type=jnp.float32)


def _scale_body(degp_ref, xt_ref, y_ref, dis_ref):
    deg = degp_ref[0, :, :1] + degp_ref[1, :, :1] + 1.0
    dis = lax.rsqrt(deg)
    dis_ref[...] = dis
    y_ref[...] = xt_ref[...] * dis


def _mid_body(agg_ref, y_ref, dis_ref, b_ref, w_ref, x1_ref, y2_ref):
    dis = dis_ref[...]
    s = agg_ref[0] + agg_ref[1] + y_ref[...]
    x1 = jnp.tanh(s * dis + b_ref[...])
    x1_ref[...] = x1
    y2_ref[...] = jnp.dot(x1, w_ref[...],
                          preferred_element_type=jnp.float32) * dis


def _final_body(agg_ref, y_ref, dis_ref, b_ref, batch_ref, x2_ref, mean_ref,
                cnt_ref):
    i = pl.program_id(0)

    @pl.when(i == 0)
    def _():
        mean_ref[...] = jnp.zeros_like(mean_ref)
        cnt_ref[...] = jnp.zeros_like(cnt_ref)

    s = agg_ref[0] + agg_ref[1] + y_ref[...]
    x2 = jnp.tanh(s * dis_ref[...] + b_ref[...])
    x2_ref[...] = x2
    onehot = (batch_ref[...] == lax.broadcasted_iota(
        jnp.int32, (_BN, _G), 1)).astype(jnp.float32)
    mean_ref[...] += lax.dot_general(onehot, x2, (((0,), (0,)), ((), ())),
                                     preferred_element_type=jnp.float32)
    cnt_ref[...] += lax.dot_general(onehot, jnp.ones((_BN, 1), jnp.float32),
                                    (((0,), (0,)), ((), ())),
                                    preferred_element_type=jnp.float32)

    @pl.when(i == _GRID - 1)
    def _():
        mean_ref[...] = mean_ref[...] / jnp.maximum(cnt_ref[...], 1.0)


def _tc_matmul1(attrs, W1):
    return pl.pallas_call(
        _mm1_body,
        grid=(_GRID,),
        in_specs=[
            pl.BlockSpec((_BN, _DIN), lambda i: (i, 0)),
            pl.BlockSpec((_DIN, _H), lambda i: (0, 0)),
        ],
        out_specs=pl.BlockSpec((_BN, _H), lambda i: (i, 0)),
        out_shape=jax.ShapeDtypeStruct((_N, _H), jnp.float32),
    )(attrs, W1)


def _tc_scale(deg_p, xt1):
    return pl.pallas_call(
        _scale_body,
        grid=(_GRID,),
        in_specs=[
            pl.BlockSpec((_NC, _BN, 16), lambda i: (0, i, 0)),
            pl.BlockSpec((_BN, _H), lambda i: (i, 0)),
        ],
        out_specs=[
            pl.BlockSpec((_BN, _H), lambda i: (i, 0)),
            pl.BlockSpec((_BN, 1), lambda i: (i, 0)),
        ],
        out_shape=[
            jax.ShapeDtypeStruct((_N, _H), jnp.float32),
            jax.ShapeDtypeStruct((_N, 1), jnp.float32),
        ],
    )(deg_p, xt1)


def _tc_mid(agg, y1, dis, b1r, W2):
    return pl.pallas_call(
        _mid_body,
        grid=(_GRID,),
        in_specs=[
            pl.BlockSpec((_NC, _BN, _H), lambda i: (0, i, 0)),
            pl.BlockSpec((_BN, _H), lambda i: (i, 0)),
            pl.BlockSpec((_BN, 1), lambda i: (i, 0)),
            pl.BlockSpec((1, _H), lambda i: (0, 0)),
            pl.BlockSpec((_H, _H), lambda i: (0, 0)),
        ],
        out_specs=[
            pl.BlockSpec((_BN, _H), lambda i: (i, 0)),
            pl.BlockSpec((_BN, _H), lambda i: (i, 0)),
        ],
        out_shape=[
            jax.ShapeDtypeStruct((_N, _H), jnp.float32),
            jax.ShapeDtypeStruct((_N, _H), jnp.float32),
        ],
    )(agg, y1, dis, b1r, W2)


def _tc_final(agg, y2, dis, b2r, batch2):
    return pl.pallas_call(
        _final_body,
        grid=(_GRID,),
        in_specs=[
            pl.BlockSpec((_NC, _BN, _H), lambda i: (0, i, 0)),
            pl.BlockSpec((_BN, _H), lambda i: (i, 0)),
            pl.BlockSpec((_BN, 1), lambda i: (i, 0)),
            pl.BlockSpec((1, _H), lambda i: (0, 0)),
            pl.BlockSpec((_BN, 1), lambda i: (i, 0)),
        ],
        out_specs=[
            pl.BlockSpec((_BN, _H), lambda i: (i, 0)),
            pl.BlockSpec((_G, _H), lambda i: (0, 0)),
        ],
        out_shape=[
            jax.ShapeDtypeStruct((_N, _H), jnp.float32),
            jax.ShapeDtypeStruct((_G, _H), jnp.float32),
        ],
        scratch_shapes=[pltpu.VMEM((_G, 1), jnp.float32)],
    )(agg, y2, dis, b2r, batch2)


def kernel(attrs, edge_index, batch, W1, b1, W2, b2):
    if "sc" not in _built:
        _built["sc"] = _build_sc_kernels()
    edge_agg, deg_hist = _built["sc"]

    f32 = jnp.float32
    src3 = edge_index[0].reshape(_NW, _NCH, _CH)
    dst3 = edge_index[1].reshape(_NW, _NCH, _CH)
    zero64 = jnp.zeros((_RPT, _H), f32)
    zero16 = jnp.zeros((_RPT, 16), f32)
    ones16 = jnp.ones((_CH, 16), f32)
    b1r = b1.reshape(1, _H)
    b2r = b2.reshape(1, _H)
    batch2 = batch.reshape(_N, 1)

    deg_p = deg_hist(dst3, zero16, ones16)
    xt1 = _tc_matmul1(attrs.astype(f32), W1)
    y1, dis = _tc_scale(deg_p, xt1)
    agg1 = edge_agg(y1, src3, dst3, zero64)
    x1, y2 = _tc_mid(agg1, y1, dis, b1r, W2)
    agg2 = edge_agg(y2, src3, dst3, zero64)
    x2, x_mean = _tc_final(agg2, y2, dis, b2r, batch2)
    return (x2, x_mean, x1, x2)
